# baseline (speedup 1.0000x reference)
import functools

import jax
import jax.numpy as jnp
from jax import lax
from jax.experimental import pallas as pl
from jax.experimental.pallas import tpu as pltpu
from jax.experimental.pallas import tpu_sc as plsc

NUM_ENTITIES = 1000000
NUM_RELATIONS = 1000
EMBED_DIM = 64
BATCH = 16384

NC = 2
NS = 16
NW = NC * NS
BPW = BATCH // NW
CHUNK = 128
NCHUNK = BPW // CHUNK


def _make_sc_lookup():
  mesh = plsc.VectorSubcoreMesh(core_axis_name="c", subcore_axis_name="s")

  @functools.partial(
      pl.kernel,
      mesh=mesh,
      compiler_params=pltpu.CompilerParams(use_tc_tiling_on_sc=False),
      out_type=(
          jax.ShapeDtypeStruct((BATCH, EMBED_DIM), jnp.float32),
          jax.ShapeDtypeStruct((BATCH, EMBED_DIM), jnp.float32),
      ),
      scratch_types=[
          pltpu.VMEM((NCHUNK, CHUNK), jnp.int32),
          pltpu.VMEM((NCHUNK, CHUNK), jnp.int32),
          pltpu.VMEM((BPW, EMBED_DIM), jnp.float32),
          pltpu.VMEM((BPW, EMBED_DIM), jnp.float32),
          pltpu.SemaphoreType.DMA,
          pltpu.SemaphoreType.DMA,
      ],
  )
  def lookup(heads_hbm, rels_hbm, ent_hbm, rel_hbm, out_h, out_r,
             hidx, ridx, hrows, rrows, hsem, rsem):
    wid = lax.axis_index("s") * NC + lax.axis_index("c")
    base = wid * BPW
    pltpu.sync_copy(heads_hbm.at[wid], hidx)
    pltpu.sync_copy(rels_hbm.at[wid], ridx)
    hcopies = [
        pltpu.async_copy(ent_hbm.at[hidx.at[j]],
                         hrows.at[pl.ds(j * CHUNK, CHUNK)], hsem)
        for j in range(NCHUNK)
    ]
    rcopies = [
        pltpu.async_copy(rel_hbm.at[ridx.at[j]],
                         rrows.at[pl.ds(j * CHUNK, CHUNK)], rsem)
        for j in range(NCHUNK)
    ]
    for c in hcopies:
      c.wait()
    pltpu.sync_copy(hrows, out_h.at[pl.ds(base, BPW)])
    for c in rcopies:
      c.wait()
    pltpu.sync_copy(rrows, out_r.at[pl.ds(base, BPW)])

  return lookup


_lookup = _make_sc_lookup()


@jax.jit
def kernel(heads, relations, entity_table, relation_table):
  heads_r = heads.astype(jnp.int32).reshape(NW, NCHUNK, CHUNK)
  rels_r = relations.astype(jnp.int32).reshape(NW, NCHUNK, CHUNK)
  return _lookup(heads_r, rels_r, entity_table, relation_table)
